# Initial kernel scaffold; baseline (speedup 1.0000x reference)
import jax
import jax.numpy as jnp
from jax.experimental import pallas as pl


def kernel(x, rating_matrix, qtus, weight, b_user, b_item):
    raise NotImplementedError("write your pallas kernel here")


# speedup vs baseline: 2.9139x; 2.9139x over previous
import functools

import jax
import jax.numpy as jnp
from jax import lax
from jax.experimental import pallas as pl
from jax.experimental.pallas import tpu as pltpu
from jax.experimental.pallas import tpu_sc as plsc

U = 1024
I = 1000
K = 50
B = 4096
NEIGH = 50.0

NC = 2
NS = 16
L = 16
NW = NC * NS
BW = B // NW
G = BW // L


def _sc_body(u_hbm, t_hbm, rt_hbm, wt_hbm, qtus_hbm, bu_hbm, bi_hbm,
             out_hbm,
             u_seg, t_seg, rid_seg, uix, tix, ridix,
             qtu_buf, rt_buf, wt_buf, bu_buf, bi_buf, out_buf,
             sem0, sem1, sem2):
    cid = lax.axis_index("c")
    sid = lax.axis_index("s")
    wid = sid * NC + cid
    base = pl.multiple_of(wid * BW, BW)

    pltpu.sync_copy(u_hbm.at[pl.ds(base, BW)], u_seg)
    pltpu.sync_copy(t_hbm.at[pl.ds(base, BW)], t_seg)
    pltpu.sync_copy(bu_hbm, bu_buf)
    pltpu.sync_copy(bi_hbm, bi_buf)

    for i in range(G):
        u16 = u_seg[pl.ds(i * L, L)]
        t16 = t_seg[pl.ds(i * L, L)]
        rid_seg[pl.ds(i * L, L)] = u16 * I + t16

    lanes = lax.iota(jnp.int32, L)

    def group(g, carry):
        off = pl.multiple_of(g * L, L)
        uix[...] = u_seg[pl.ds(off, L)]
        tix[...] = t_seg[pl.ds(off, L)]
        rid16 = rid_seg[pl.ds(off, L)]
        ridix[...] = lax.shift_right_logical(rid16, 2)
        sub16 = (rid16 & 3) * K
        cp0 = pltpu.async_copy(qtus_hbm.at[ridix], qtu_buf, sem0)
        cp1 = pltpu.async_copy(rt_hbm.at[uix], rt_buf, sem1)
        cp2 = pltpu.async_copy(wt_hbm.at[tix], wt_buf, sem2)
        cp0.wait()
        cp1.wait()
        cp2.wait()

        u16 = uix[...]
        t16 = tix[...]
        bu_v = plsc.load_gather(bu_buf, [u16])
        bt_v = plsc.load_gather(bi_buf, [t16])

        def kstep(k, acc):
            kcol = sub16 + k
            j = plsc.load_gather(qtu_buf, [lanes, kcol])
            wv = plsc.load_gather(wt_buf, [lanes, j])
            rv = plsc.load_gather(rt_buf, [lanes, j])
            biv = plsc.load_gather(bi_buf, [j])
            return acc + wv * (rv - bu_v - biv)

        acc = lax.fori_loop(0, K, kstep, jnp.zeros((L,), jnp.float32))
        out_buf[pl.ds(off, L)] = bu_v + bt_v + acc / NEIGH
        return carry

    lax.fori_loop(0, G, group, 0)
    pltpu.sync_copy(out_buf, out_hbm.at[pl.ds(base, BW)])


@functools.partial(jax.jit)
def _sc_call(u, t, rating_matrix, wt, qtus2d, b_user, b_item):
    mesh = plsc.VectorSubcoreMesh(core_axis_name="c", subcore_axis_name="s")
    f = pl.kernel(
        _sc_body,
        out_type=jax.ShapeDtypeStruct((B,), jnp.float32),
        mesh=mesh,
        scratch_types=[
            pltpu.VMEM((BW,), jnp.int32),
            pltpu.VMEM((BW,), jnp.int32),
            pltpu.VMEM((BW,), jnp.int32),
            pltpu.VMEM((L,), jnp.int32),
            pltpu.VMEM((L,), jnp.int32),
            pltpu.VMEM((L,), jnp.int32),
            pltpu.VMEM((L, 4 * K), jnp.int32),
            pltpu.VMEM((L, I), jnp.float32),
            pltpu.VMEM((L, I), jnp.float32),
            pltpu.VMEM((U,), jnp.float32),
            pltpu.VMEM((I,), jnp.float32),
            pltpu.VMEM((BW,), jnp.float32),
            pltpu.SemaphoreType.DMA,
            pltpu.SemaphoreType.DMA,
            pltpu.SemaphoreType.DMA,
        ],
        compiler_params=pltpu.CompilerParams(
            needs_layout_passes=False, use_tc_tiling_on_sc=False),
    )
    return f(u, t, rating_matrix, wt, qtus2d, b_user, b_item)


def kernel(x, rating_matrix, qtus, weight, b_user, b_item):
    u = x[0]
    t = x[1]
    wt = weight.T
    qtus2d = qtus.reshape(U * I // 4, 4 * K)
    return _sc_call(u, t, rating_matrix, wt, qtus2d, b_user, b_item)
